# baseline (speedup 1.0000x reference)
import functools

import jax
import jax.numpy as jnp
from jax import lax
from jax.experimental import pallas as pl
from jax.experimental.pallas import tpu as pltpu
from jax.experimental.pallas import tpu_sc as plsc

N = 100000
E = 1600000
LOGBLK = 13
BLK = 1 << LOGBLK
NB = 13
NPAD = NB * BLK
DUMP = BLK
ACC_ROWS = 8448
SRC_BITS = 17
SENT = DUMP << SRC_BITS
NCHUNK = 250
CHUNK_E = 6400
STEPS = CHUNK_E // 16
STG = CHUNK_E + 16
EDGE_CAP = 1703936
UNIT = 128
NW = 32
R = 2000

_I16 = lambda: lax.iota(jnp.int32, 16)


def _vext(vec, i):
    return jnp.max(jnp.where(_I16() == i, vec, 0))


def _take16(vec, idx):
    return vec.at[idx].get(mode="promise_in_bounds")


def _seg_info(sk):
    iota = _I16()
    prev = _take16(sk, jnp.maximum(iota - 1, 0))
    change = jnp.logical_or(sk != prev, iota == 0)
    first = plsc.cummax(jnp.where(change, iota, 0))
    rank = iota - first
    nxt = _take16(sk, jnp.minimum(iota + 1, 15))
    ends = jnp.logical_or(sk != nxt, iota == 15)
    return rank, ends


def _mesh():
    return plsc.VectorSubcoreMesh(core_axis_name="c", subcore_axis_name="s")


def _wid():
    return lax.axis_index("s") * 2 + lax.axis_index("c")


def _ds8(off, size):
    return pl.ds(pl.multiple_of(off, 8), size)


@functools.cache
def _k_count():
    tile_rows = NPAD // 16

    def body(dst_hbm, cnts_hbm, degc_hbm, dbuf, crow, ones, zbuf, accs):
        wid = _wid()
        core = lax.axis_index("c")
        sub = lax.axis_index("s")

        def f1(i, _):
            ones[pl.ds(i * 16, 16)] = jnp.ones((16,), jnp.float32)
            return 0

        lax.fori_loop(0, CHUNK_E // 16, f1, 0)

        def f0(i, _):
            zbuf[pl.ds(i * 16, 16)] = jnp.zeros((16,), jnp.float32)
            return 0

        lax.fori_loop(0, tile_rows // 16, f0, 0)
        pltpu.sync_copy(zbuf, accs.at[_ds8(sub * tile_rows, tile_rows)])
        plsc.subcore_barrier()

        def chunk_iter(i, _):
            chunk = wid + NW * i

            @pl.when(chunk < NCHUNK)
            def _():
                pltpu.sync_copy(dst_hbm.at[_ds8(chunk * CHUNK_E, CHUNK_E)], dbuf)
                crow[...] = jnp.zeros((16,), jnp.int32)

                def step(j, _):
                    v = dbuf[pl.ds(j * 16, 16)]
                    bkt = lax.shift_right_logical(v, LOGBLK)
                    sk, _sv = plsc.sort_key_val(bkt, bkt)
                    rank, ends = _seg_info(sk)
                    pos = _take16(crow[...], sk) + rank
                    plsc.store_scatter(crow, [sk], pos + 1, mask=ends)
                    return 0

                lax.fori_loop(0, STEPS, step, 0)
                pltpu.sync_copy(crow, cnts_hbm.at[_ds8(chunk * 16, 16)])
                pltpu.sync_copy(ones, accs.at[dbuf], add=True)

            return 0

        lax.fori_loop(0, (NCHUNK + NW - 1) // NW, chunk_iter, 0)
        plsc.subcore_barrier()
        pltpu.sync_copy(
            accs.at[_ds8(sub * tile_rows, tile_rows)],
            degc_hbm.at[core, _ds8(sub * tile_rows, tile_rows)])

    return pl.kernel(
        body,
        out_type=(
            jax.ShapeDtypeStruct((NCHUNK * 16,), jnp.int32),
            jax.ShapeDtypeStruct((2, NPAD), jnp.float32),
        ),
        mesh=_mesh(),
        compiler_params=pltpu.CompilerParams(needs_layout_passes=False, use_tc_tiling_on_sc=False),
        scratch_types=[
            pltpu.VMEM((CHUNK_E,), jnp.int32),
            pltpu.VMEM((16,), jnp.int32),
            pltpu.VMEM((CHUNK_E,), jnp.float32),
            pltpu.VMEM((tile_rows,), jnp.float32),
            pltpu.VMEM_SHARED((NPAD,), jnp.float32),
        ],
    )


@functools.cache
def _k_prefix():
    def body(cnts_hbm, starts_hbm, bstart_hbm, braw_hbm, cbuf, sbuf, vb):
        wid = _wid()

        @pl.when(wid == 0)
        def _():
            pltpu.sync_copy(cnts_hbm, cbuf)

            def p1(i, run):
                row = cbuf[pl.ds(i * 16, 16)]
                r8 = jnp.bitwise_and(row + 7, jnp.int32(-8))
                sbuf[pl.ds(i * 16, 16)] = run
                return run + r8

            traw = lax.fori_loop(0, NCHUNK, p1, jnp.zeros((16,), jnp.int32))
            tp = jnp.bitwise_and(traw + 2047, jnp.int32(-2048))
            bex = plsc.cumsum(tp) - tp

            def p2(i, _):
                sbuf[pl.ds(i * 16, 16)] = sbuf[pl.ds(i * 16, 16)] + bex
                return 0

            lax.fori_loop(0, NCHUNK, p2, 0)
            pltpu.sync_copy(sbuf, starts_hbm)
            vb[...] = bex
            pltpu.sync_copy(vb, bstart_hbm)
            vb[...] = bex + traw
            pltpu.sync_copy(vb, braw_hbm)

    return pl.kernel(
        body,
        out_type=(
            jax.ShapeDtypeStruct((NCHUNK * 16,), jnp.int32),
            jax.ShapeDtypeStruct((16,), jnp.int32),
            jax.ShapeDtypeStruct((16,), jnp.int32),
        ),
        mesh=_mesh(),
        compiler_params=pltpu.CompilerParams(needs_layout_passes=False, use_tc_tiling_on_sc=False),
        scratch_types=[
            pltpu.VMEM((NCHUNK * 16,), jnp.int32),
            pltpu.VMEM((NCHUNK * 16,), jnp.int32),
            pltpu.VMEM((16,), jnp.int32),
        ],
    )


_FLUSH_SIZES = (1024, 512, 256, 128, 64, 32, 16, 8)


@functools.cache
def _k_scatter():
    def body(src_hbm, dst_hbm, starts_hbm, bstart_hbm, braw_hbm, edge_hbm,
             sbuf, dbuf, stg, strow, cntref, sentbuf):
        wid = _wid()
        iota = _I16()
        sent = jnp.full((16,), SENT, jnp.int32)

        def fs(i, _):
            sentbuf[pl.ds(i * 16, 16)] = sent
            return 0

        lax.fori_loop(0, 64, fs, 0)

        def chunk_iter(i, _):
            chunk = wid + NW * i

            @pl.when(chunk < NCHUNK)
            def _():
                pltpu.sync_copy(src_hbm.at[_ds8(chunk * CHUNK_E, CHUNK_E)], sbuf)
                pltpu.sync_copy(dst_hbm.at[_ds8(chunk * CHUNK_E, CHUNK_E)], dbuf)
                pltpu.sync_copy(starts_hbm.at[_ds8(chunk * 16, 16)], strow)
                cntref[...] = jnp.zeros((16,), jnp.int32)

                def step(j, _):
                    vs = sbuf[pl.ds(j * 16, 16)]
                    vd = dbuf[pl.ds(j * 16, 16)]
                    bkt = lax.shift_right_logical(vd, LOGBLK)
                    dstl = jnp.bitwise_and(vd, BLK - 1)
                    packed = jnp.bitwise_or(vs, lax.shift_left(dstl, SRC_BITS))
                    sk, sv = plsc.sort_key_val(bkt, packed)
                    rank, ends = _seg_info(sk)
                    pos = _take16(cntref[...], sk) + rank
                    plsc.store_scatter(stg, [sk * STG + pos], sv)
                    plsc.store_scatter(cntref, [sk], pos + 1, mask=ends)
                    return 0

                lax.fori_loop(0, STEPS, step, 0)
                cntv = cntref[...]
                stv = strow[...]
                for b in range(NB):
                    cnt = _vext(cntv, b)
                    start = _vext(stv, b)
                    plsc.store_scatter(stg, [b * STG + cnt + iota], sent)
                    f = jnp.bitwise_and(cnt + 7, jnp.int32(-8))
                    n2k = lax.shift_right_logical(f, 11)
                    rem = jnp.bitwise_and(f, 2047)
                    base = lax.shift_left(n2k, 11)

                    def fl(t, _, b=b, start=start):
                        pltpu.sync_copy(
                            stg.at[_ds8(b * STG + t * 2048, 2048)],
                            edge_hbm.at[_ds8(start + t * 2048, 2048)])
                        return 0

                    lax.fori_loop(0, n2k, fl, 0)
                    for sz in _FLUSH_SIZES:
                        off = base + jnp.bitwise_and(rem, jnp.int32(~(2 * sz - 1)))

                        @pl.when(jnp.bitwise_and(rem, sz) != 0)
                        def _(off=off, sz=sz, b=b, start=start):
                            pltpu.sync_copy(
                                stg.at[_ds8(b * STG + off, sz)],
                                edge_hbm.at[_ds8(start + off, sz)])

            return 0

        lax.fori_loop(0, (NCHUNK + NW - 1) // NW, chunk_iter, 0)

        @pl.when(wid < NB)
        def _():
            pltpu.sync_copy(braw_hbm, strow)
            braw_v = strow[...]
            pltpu.sync_copy(bstart_hbm, cntref)
            bs_v = cntref[...]
            gstart = _vext(braw_v, wid)
            gend = _vext(bs_v, wid + 1)
            gap = gend - gstart
            for sz in _FLUSH_SIZES:
                off = gstart + jnp.bitwise_and(gap, jnp.int32(~(2 * sz - 1)))

                @pl.when(jnp.bitwise_and(gap, sz) != 0)
                def _(off=off, sz=sz):
                    pltpu.sync_copy(sentbuf.at[pl.ds(0, sz)],
                                    edge_hbm.at[_ds8(off, sz)])

    return pl.kernel(
        body,
        out_type=jax.ShapeDtypeStruct((EDGE_CAP,), jnp.int32),
        mesh=_mesh(),
        compiler_params=pltpu.CompilerParams(needs_layout_passes=False, use_tc_tiling_on_sc=False),
        scratch_types=[
            pltpu.VMEM((CHUNK_E,), jnp.int32),
            pltpu.VMEM((CHUNK_E,), jnp.int32),
            pltpu.VMEM((NB * STG,), jnp.int32),
            pltpu.VMEM((16,), jnp.int32),
            pltpu.VMEM((16,), jnp.int32),
            pltpu.VMEM((1024,), jnp.int32),
        ],
    )


@functools.cache
def _k_prop(w):
    unit = {16: 1024, 64: 512, 128: 128}[w]
    lgu = unit.bit_length() - 1
    nsteps = unit // 16

    def body(edge_hbm, bstart_hbm, y_hbm, zeros_hbm, acc_hbm,
             bsv, eb0, eb1, si0, si1, di0, di1, st0, st1, accs,
             sem0, sem1, esem0, esem1, ssem0, ssem1):
        core = lax.axis_index("c")
        s = lax.axis_index("s")
        pltpu.sync_copy(bstart_hbm, bsv)
        bvec = bsv[...]
        ebs, sis, dis, sts = (eb0, eb1), (si0, si1), (di0, di1), (st0, st1)
        sems, esems, ssems = (sem0, sem1), (esem0, esem1), (ssem0, ssem1)

        def blk_iter(i, _):
            b = core + 2 * i

            @pl.when(b < NB)
            def _():
                pltpu.sync_copy(zeros_hbm, accs.at[_ds8(s * 528, 528)])
                plsc.subcore_barrier()
                ustart = _vext(bvec, b)
                uend = _vext(bvec, b + 1)
                nu = lax.shift_right_logical(uend - ustart, lgu)
                nk = lax.shift_right_logical(nu - s + 15, 4)

                def eaddr(k):
                    return _ds8(ustart + lax.shift_left(s + 16 * k, lgu), unit)

                def unpack(ph):
                    def up(m, _):
                        v = ebs[ph][pl.ds(m * 16, 16)]
                        sis[ph][pl.ds(m * 16, 16)] = jnp.bitwise_and(
                            v, jnp.int32((1 << SRC_BITS) - 1))
                        dis[ph][pl.ds(m * 16, 16)] = lax.shift_right_logical(
                            v, SRC_BITS)
                        return 0

                    lax.fori_loop(0, nsteps, up, 0)

                def start_gather(ph):
                    pltpu.make_async_copy(
                        y_hbm.at[sis[ph]], sts[ph], sems[ph]).start()

                def wait_scatter(ph):
                    pltpu.make_async_copy(
                        sts[ph], accs.at[dis[ph]], ssems[ph]).wait()

                @pl.when(nk > 0)
                def _():
                    pltpu.sync_copy(edge_hbm.at[eaddr(0)], ebs[0])
                    unpack(0)
                    start_gather(0)

                @pl.when(nk > 1)
                def _():
                    pltpu.make_async_copy(
                        edge_hbm.at[eaddr(1)], ebs[1], esems[1]).start()

                def pair(k2, _):
                    for ph in range(2):
                        k = 2 * k2 + ph

                        @pl.when(k < nk)
                        def _(k=k, ph=ph):
                            @pl.when(k + 2 < nk)
                            def _():
                                pltpu.make_async_copy(
                                    edge_hbm.at[eaddr(k + 2)], ebs[ph],
                                    esems[ph]).start()

                            @pl.when(k + 1 < nk)
                            def _():
                                pltpu.make_async_copy(
                                    edge_hbm.at[eaddr(k + 1)], ebs[1 - ph],
                                    esems[1 - ph]).wait()

                                @pl.when(k >= 1)
                                def _():
                                    wait_scatter(1 - ph)

                                unpack(1 - ph)
                                start_gather(1 - ph)

                            pltpu.make_async_copy(
                                y_hbm.at[sis[ph]], sts[ph], sems[ph]).wait()
                            pltpu.make_async_copy(
                                sts[ph], accs.at[dis[ph]],
                                ssems[ph]).start(add=True)
                    return 0

                lax.fori_loop(0, lax.shift_right_logical(nk + 1, 1), pair, 0)
                for par in range(2):
                    @pl.when((nk > 0) & (jnp.bitwise_and(nk - 1, 1) == par))
                    def _(par=par):
                        wait_scatter(par)

                    @pl.when((nk > 1) & (jnp.bitwise_and(nk, 1) == par))
                    def _(par=par):
                        wait_scatter(par)
                plsc.subcore_barrier()
                pltpu.sync_copy(accs.at[_ds8(s * 512, 512)],
                                acc_hbm.at[_ds8(b * BLK + s * 512, 512)])
                plsc.subcore_barrier()

            return 0

        lax.fori_loop(0, (NB + 1) // 2, blk_iter, 0)

    return pl.kernel(
        body,
        out_type=jax.ShapeDtypeStruct((NPAD, w), jnp.float32),
        mesh=_mesh(),
        compiler_params=pltpu.CompilerParams(needs_layout_passes=False, use_tc_tiling_on_sc=False),
        scratch_types=[
            pltpu.VMEM((16,), jnp.int32),
            pltpu.VMEM((unit,), jnp.int32),
            pltpu.VMEM((unit,), jnp.int32),
            pltpu.VMEM((unit,), jnp.int32),
            pltpu.VMEM((unit,), jnp.int32),
            pltpu.VMEM((unit,), jnp.int32),
            pltpu.VMEM((unit,), jnp.int32),
            pltpu.VMEM((unit, w), jnp.float32),
            pltpu.VMEM((unit, w), jnp.float32),
            pltpu.VMEM_SHARED((ACC_ROWS, w), jnp.float32),
            pltpu.SemaphoreType.DMA,
            pltpu.SemaphoreType.DMA,
            pltpu.SemaphoreType.DMA,
            pltpu.SemaphoreType.DMA,
            pltpu.SemaphoreType.DMA,
            pltpu.SemaphoreType.DMA,
        ],
    )


def _row_spec(c):
    return pl.BlockSpec((R, c), lambda i: (i, 0))


def _full_spec(shape):
    return pl.BlockSpec(shape, lambda i: tuple(0 for _ in shape))


@functools.cache
def _t0():
    def body(x_ref, da_ref, db_ref, y_ref, dv_ref):
        dv = 1.0 / jnp.sqrt(da_ref[...] + db_ref[...] + 1.0)
        y_ref[...] = x_ref[...] * dv
        dv_ref[...] = dv

    return pl.pallas_call(
        body,
        grid=(N // R,),
        in_specs=[_row_spec(16), _row_spec(1), _row_spec(1)],
        out_specs=[_row_spec(16), _row_spec(1)],
        out_shape=[
            jax.ShapeDtypeStruct((N, 16), jnp.float32),
            jax.ShapeDtypeStruct((N, 1), jnp.float32),
        ],
    )


@functools.cache
def _t_layer(cin, cout):
    def body(a_ref, y_ref, dv_ref, w_ref, b_ref, o_ref):
        dv = dv_ref[...]
        g = dv * (a_ref[...] + y_ref[...])
        h = jnp.maximum(jnp.dot(g, w_ref[...],
                                preferred_element_type=jnp.float32)
                        + b_ref[...], 0.0)
        o_ref[...] = dv * h

    return pl.pallas_call(
        body,
        grid=(N // R,),
        in_specs=[_row_spec(cin), _row_spec(cin), _row_spec(1),
                  _full_spec((cin, cout)), _full_spec((1, cout))],
        out_specs=_row_spec(cout),
        out_shape=jax.ShapeDtypeStruct((N, cout), jnp.float32),
    )


@functools.cache
def _t3():
    def body(a_ref, y_ref, dv_ref, w3_ref, b3_ref, wf1_ref, bf1_ref,
             wf2_ref, bf2_ref, o_ref):
        dv = dv_ref[...]
        g = dv * (a_ref[...] + y_ref[...])
        h3 = jnp.maximum(jnp.dot(g, w3_ref[...],
                                 preferred_element_type=jnp.float32)
                         + b3_ref[...], 0.0)
        t = jnp.maximum(jnp.dot(h3, wf1_ref[...],
                                preferred_element_type=jnp.float32)
                        + bf1_ref[...], 0.0)
        o_ref[...] = jnp.dot(t, wf2_ref[...],
                             preferred_element_type=jnp.float32) + bf2_ref[...]

    return pl.pallas_call(
        body,
        grid=(N // R,),
        in_specs=[_row_spec(128), _row_spec(128), _row_spec(1),
                  _full_spec((128, 128)), _full_spec((1, 128)),
                  _full_spec((128, 64)), _full_spec((1, 64)),
                  _full_spec((64, 2)), _full_spec((1, 2))],
        out_specs=_row_spec(2),
        out_shape=jax.ShapeDtypeStruct((N, 2), jnp.float32),
    )


def kernel(x, edge_index, W1, b1, W2, b2, W3, b3, Wf1, bf1, Wf2, bf2):
    edge_index = edge_index.astype(jnp.int32)
    src = edge_index[0]
    dst = edge_index[1]
    xp = jnp.pad(x, ((0, 0), (0, 13)))
    W1p = jnp.pad(W1, ((0, 13), (0, 0)))

    cnts, degc2 = _k_count()(dst)
    starts, bstart, braw = _k_prefix()(cnts)
    edge_buf = _k_scatter()(src, dst, starts, bstart, braw)

    y0, dinv = _t0()(xp, degc2[0, :N, None], degc2[1, :N, None])
    acc0 = _k_prop(16)(edge_buf, bstart, y0, jnp.zeros((528, 16), jnp.float32))
    y1 = _t_layer(16, 64)(acc0[:N], y0, dinv, W1p, b1[None])
    acc1 = _k_prop(64)(edge_buf, bstart, y1, jnp.zeros((528, 64), jnp.float32))
    y2 = _t_layer(64, 128)(acc1[:N], y1, dinv, W2, b2[None])
    acc2 = _k_prop(128)(edge_buf, bstart, y2,
                        jnp.zeros((528, 128), jnp.float32))
    out = _t3()(acc2[:N], y2, dinv, W3, b3[None], Wf1, bf1[None],
                Wf2, bf2[None])
    return out


# speedup vs baseline: 1.0053x; 1.0053x over previous
import functools

import jax
import jax.numpy as jnp
from jax import lax
from jax.experimental import pallas as pl
from jax.experimental.pallas import tpu as pltpu
from jax.experimental.pallas import tpu_sc as plsc

N = 100000
E = 1600000
LOGBLK = 13
BLK = 1 << LOGBLK
NB = 13
NPAD = NB * BLK
DUMP = BLK
ACC_ROWS = 8448
SRC_BITS = 17
SENT = DUMP << SRC_BITS
NCHUNK = 250
CHUNK_E = 6400
STEPS = CHUNK_E // 16
STG = CHUNK_E + 16
EDGE_CAP = 1703936
UNIT = 128
NW = 32
R = 2000

_I16 = lambda: lax.iota(jnp.int32, 16)


def _vext(vec, i):
    return jnp.max(jnp.where(_I16() == i, vec, 0))


def _take16(vec, idx):
    return vec.at[idx].get(mode="promise_in_bounds")


def _seg_info(sk):
    iota = _I16()
    prev = _take16(sk, jnp.maximum(iota - 1, 0))
    change = jnp.logical_or(sk != prev, iota == 0)
    first = plsc.cummax(jnp.where(change, iota, 0))
    rank = iota - first
    nxt = _take16(sk, jnp.minimum(iota + 1, 15))
    ends = jnp.logical_or(sk != nxt, iota == 15)
    return rank, ends


def _mesh():
    return plsc.VectorSubcoreMesh(core_axis_name="c", subcore_axis_name="s")


def _wid():
    return lax.axis_index("s") * 2 + lax.axis_index("c")


def _ds8(off, size):
    return pl.ds(pl.multiple_of(off, 8), size)


@functools.cache
def _k_count():
    tile_rows = NPAD // 16

    def body(dst_hbm, cnts_hbm, degc_hbm, dbuf, crow, ones, zbuf, accs):
        wid = _wid()
        core = lax.axis_index("c")
        sub = lax.axis_index("s")

        def f1(i, _):
            ones[pl.ds(i * 16, 16)] = jnp.ones((16,), jnp.float32)
            return 0

        lax.fori_loop(0, CHUNK_E // 16, f1, 0)

        def f0(i, _):
            zbuf[pl.ds(i * 16, 16)] = jnp.zeros((16,), jnp.float32)
            return 0

        lax.fori_loop(0, tile_rows // 16, f0, 0)
        pltpu.sync_copy(zbuf, accs.at[_ds8(sub * tile_rows, tile_rows)])
        plsc.subcore_barrier()

        def chunk_iter(i, _):
            chunk = wid + NW * i

            @pl.when(chunk < NCHUNK)
            def _():
                pltpu.sync_copy(dst_hbm.at[_ds8(chunk * CHUNK_E, CHUNK_E)], dbuf)
                crow[...] = jnp.zeros((16,), jnp.int32)

                def step(j, _):
                    v = dbuf[pl.ds(j * 16, 16)]
                    bkt = lax.shift_right_logical(v, LOGBLK)
                    sk, _sv = plsc.sort_key_val(bkt, bkt)
                    rank, ends = _seg_info(sk)
                    pos = _take16(crow[...], sk) + rank
                    plsc.store_scatter(crow, [sk], pos + 1, mask=ends)
                    return 0

                lax.fori_loop(0, STEPS, step, 0)
                pltpu.sync_copy(crow, cnts_hbm.at[_ds8(chunk * 16, 16)])
                pltpu.sync_copy(ones, accs.at[dbuf], add=True)

            return 0

        lax.fori_loop(0, (NCHUNK + NW - 1) // NW, chunk_iter, 0)
        plsc.subcore_barrier()
        pltpu.sync_copy(
            accs.at[_ds8(sub * tile_rows, tile_rows)],
            degc_hbm.at[core, _ds8(sub * tile_rows, tile_rows)])

    return pl.kernel(
        body,
        out_type=(
            jax.ShapeDtypeStruct((NCHUNK * 16,), jnp.int32),
            jax.ShapeDtypeStruct((2, NPAD), jnp.float32),
        ),
        mesh=_mesh(),
        compiler_params=pltpu.CompilerParams(needs_layout_passes=False, use_tc_tiling_on_sc=False),
        scratch_types=[
            pltpu.VMEM((CHUNK_E,), jnp.int32),
            pltpu.VMEM((16,), jnp.int32),
            pltpu.VMEM((CHUNK_E,), jnp.float32),
            pltpu.VMEM((tile_rows,), jnp.float32),
            pltpu.VMEM_SHARED((NPAD,), jnp.float32),
        ],
    )


@functools.cache
def _k_prefix():
    def body(cnts_hbm, starts_hbm, bstart_hbm, braw_hbm, cbuf, sbuf, vb):
        wid = _wid()

        @pl.when(wid == 0)
        def _():
            pltpu.sync_copy(cnts_hbm, cbuf)

            def p1(i, run):
                row = cbuf[pl.ds(i * 16, 16)]
                r8 = jnp.bitwise_and(row + 7, jnp.int32(-8))
                sbuf[pl.ds(i * 16, 16)] = run
                return run + r8

            traw = lax.fori_loop(0, NCHUNK, p1, jnp.zeros((16,), jnp.int32))
            tp = jnp.bitwise_and(traw + 2047, jnp.int32(-2048))
            bex = plsc.cumsum(tp) - tp

            def p2(i, _):
                sbuf[pl.ds(i * 16, 16)] = sbuf[pl.ds(i * 16, 16)] + bex
                return 0

            lax.fori_loop(0, NCHUNK, p2, 0)
            pltpu.sync_copy(sbuf, starts_hbm)
            vb[...] = bex
            pltpu.sync_copy(vb, bstart_hbm)
            vb[...] = bex + traw
            pltpu.sync_copy(vb, braw_hbm)

    return pl.kernel(
        body,
        out_type=(
            jax.ShapeDtypeStruct((NCHUNK * 16,), jnp.int32),
            jax.ShapeDtypeStruct((16,), jnp.int32),
            jax.ShapeDtypeStruct((16,), jnp.int32),
        ),
        mesh=_mesh(),
        compiler_params=pltpu.CompilerParams(needs_layout_passes=False, use_tc_tiling_on_sc=False),
        scratch_types=[
            pltpu.VMEM((NCHUNK * 16,), jnp.int32),
            pltpu.VMEM((NCHUNK * 16,), jnp.int32),
            pltpu.VMEM((16,), jnp.int32),
        ],
    )


_FLUSH_SIZES = (1024, 512, 256, 128, 64, 32, 16, 8)


@functools.cache
def _k_scatter():
    def body(src_hbm, dst_hbm, starts_hbm, bstart_hbm, braw_hbm, edge_hbm,
             sbuf, dbuf, stg, strow, cntref, sentbuf):
        wid = _wid()
        iota = _I16()
        sent = jnp.full((16,), SENT, jnp.int32)

        def fs(i, _):
            sentbuf[pl.ds(i * 16, 16)] = sent
            return 0

        lax.fori_loop(0, 64, fs, 0)

        def chunk_iter(i, _):
            chunk = wid + NW * i

            @pl.when(chunk < NCHUNK)
            def _():
                pltpu.sync_copy(src_hbm.at[_ds8(chunk * CHUNK_E, CHUNK_E)], sbuf)
                pltpu.sync_copy(dst_hbm.at[_ds8(chunk * CHUNK_E, CHUNK_E)], dbuf)
                pltpu.sync_copy(starts_hbm.at[_ds8(chunk * 16, 16)], strow)
                cntref[...] = jnp.zeros((16,), jnp.int32)

                def step(j, _):
                    vs = sbuf[pl.ds(j * 16, 16)]
                    vd = dbuf[pl.ds(j * 16, 16)]
                    bkt = lax.shift_right_logical(vd, LOGBLK)
                    dstl = jnp.bitwise_and(vd, BLK - 1)
                    packed = jnp.bitwise_or(vs, lax.shift_left(dstl, SRC_BITS))
                    sk, sv = plsc.sort_key_val(bkt, packed)
                    rank, ends = _seg_info(sk)
                    pos = _take16(cntref[...], sk) + rank
                    plsc.store_scatter(stg, [sk * STG + pos], sv)
                    plsc.store_scatter(cntref, [sk], pos + 1, mask=ends)
                    return 0

                lax.fori_loop(0, STEPS, step, 0)
                cntv = cntref[...]
                stv = strow[...]
                for b in range(NB):
                    cnt = _vext(cntv, b)
                    start = _vext(stv, b)
                    plsc.store_scatter(stg, [b * STG + cnt + iota], sent)
                    f = jnp.bitwise_and(cnt + 7, jnp.int32(-8))
                    n2k = lax.shift_right_logical(f, 11)
                    rem = jnp.bitwise_and(f, 2047)
                    base = lax.shift_left(n2k, 11)

                    def fl(t, _, b=b, start=start):
                        pltpu.sync_copy(
                            stg.at[_ds8(b * STG + t * 2048, 2048)],
                            edge_hbm.at[_ds8(start + t * 2048, 2048)])
                        return 0

                    lax.fori_loop(0, n2k, fl, 0)
                    for sz in _FLUSH_SIZES:
                        off = base + jnp.bitwise_and(rem, jnp.int32(~(2 * sz - 1)))

                        @pl.when(jnp.bitwise_and(rem, sz) != 0)
                        def _(off=off, sz=sz, b=b, start=start):
                            pltpu.sync_copy(
                                stg.at[_ds8(b * STG + off, sz)],
                                edge_hbm.at[_ds8(start + off, sz)])

            return 0

        lax.fori_loop(0, (NCHUNK + NW - 1) // NW, chunk_iter, 0)

        @pl.when(wid < NB)
        def _():
            pltpu.sync_copy(braw_hbm, strow)
            braw_v = strow[...]
            pltpu.sync_copy(bstart_hbm, cntref)
            bs_v = cntref[...]
            gstart = _vext(braw_v, wid)
            gend = _vext(bs_v, wid + 1)
            gap = gend - gstart
            for sz in _FLUSH_SIZES:
                off = gstart + jnp.bitwise_and(gap, jnp.int32(~(2 * sz - 1)))

                @pl.when(jnp.bitwise_and(gap, sz) != 0)
                def _(off=off, sz=sz):
                    pltpu.sync_copy(sentbuf.at[pl.ds(0, sz)],
                                    edge_hbm.at[_ds8(off, sz)])

    return pl.kernel(
        body,
        out_type=jax.ShapeDtypeStruct((EDGE_CAP,), jnp.int32),
        mesh=_mesh(),
        compiler_params=pltpu.CompilerParams(needs_layout_passes=False, use_tc_tiling_on_sc=False),
        scratch_types=[
            pltpu.VMEM((CHUNK_E,), jnp.int32),
            pltpu.VMEM((CHUNK_E,), jnp.int32),
            pltpu.VMEM((NB * STG,), jnp.int32),
            pltpu.VMEM((16,), jnp.int32),
            pltpu.VMEM((16,), jnp.int32),
            pltpu.VMEM((1024,), jnp.int32),
        ],
    )


@functools.cache
def _k_prop(w):
    unit = max(16, min(512, 16384 // w))
    lgu = unit.bit_length() - 1
    nsteps = unit // 16

    def body(edge_hbm, bstart_hbm, y_hbm, zeros_hbm, acc_hbm,
             bsv, eb0, eb1, si0, si1, di0, di1, st0, st1, accs,
             sem0, sem1, esem0, esem1, ssem0, ssem1):
        core = lax.axis_index("c")
        s = lax.axis_index("s")
        pltpu.sync_copy(bstart_hbm, bsv)
        bvec = bsv[...]
        ebs, sis, dis, sts = (eb0, eb1), (si0, si1), (di0, di1), (st0, st1)
        sems, esems, ssems = (sem0, sem1), (esem0, esem1), (ssem0, ssem1)

        def blk_iter(i, _):
            b = core + 2 * i

            @pl.when(b < NB)
            def _():
                pltpu.sync_copy(zeros_hbm, accs.at[_ds8(s * 528, 528)])
                plsc.subcore_barrier()
                ustart = _vext(bvec, b)
                uend = _vext(bvec, b + 1)
                nu = lax.shift_right_logical(uend - ustart, lgu)
                nk = lax.shift_right_logical(nu - s + 15, 4)

                def eaddr(k):
                    return _ds8(ustart + lax.shift_left(s + 16 * k, lgu), unit)

                def unpack(ph):
                    def up(m, _):
                        v = ebs[ph][pl.ds(m * 16, 16)]
                        sis[ph][pl.ds(m * 16, 16)] = jnp.bitwise_and(
                            v, jnp.int32((1 << SRC_BITS) - 1))
                        dis[ph][pl.ds(m * 16, 16)] = lax.shift_right_logical(
                            v, SRC_BITS)
                        return 0

                    lax.fori_loop(0, nsteps, up, 0)

                def start_gather(ph):
                    pltpu.make_async_copy(
                        y_hbm.at[sis[ph]], sts[ph], sems[ph]).start()

                def wait_scatter(ph):
                    pltpu.make_async_copy(
                        sts[ph], accs.at[dis[ph]], ssems[ph]).wait()

                @pl.when(nk > 0)
                def _():
                    pltpu.sync_copy(edge_hbm.at[eaddr(0)], ebs[0])
                    unpack(0)
                    start_gather(0)

                @pl.when(nk > 1)
                def _():
                    pltpu.make_async_copy(
                        edge_hbm.at[eaddr(1)], ebs[1], esems[1]).start()

                def pair(k2, _):
                    for ph in range(2):
                        k = 2 * k2 + ph

                        @pl.when(k < nk)
                        def _(k=k, ph=ph):
                            @pl.when(k + 2 < nk)
                            def _():
                                pltpu.make_async_copy(
                                    edge_hbm.at[eaddr(k + 2)], ebs[ph],
                                    esems[ph]).start()

                            @pl.when(k + 1 < nk)
                            def _():
                                pltpu.make_async_copy(
                                    edge_hbm.at[eaddr(k + 1)], ebs[1 - ph],
                                    esems[1 - ph]).wait()

                                @pl.when(k >= 1)
                                def _():
                                    wait_scatter(1 - ph)

                                unpack(1 - ph)
                                start_gather(1 - ph)

                            pltpu.make_async_copy(
                                y_hbm.at[sis[ph]], sts[ph], sems[ph]).wait()
                            pltpu.make_async_copy(
                                sts[ph], accs.at[dis[ph]],
                                ssems[ph]).start(add=True)
                    return 0

                lax.fori_loop(0, lax.shift_right_logical(nk + 1, 1), pair, 0)
                for par in range(2):
                    @pl.when((nk > 0) & (jnp.bitwise_and(nk - 1, 1) == par))
                    def _(par=par):
                        wait_scatter(par)

                    @pl.when((nk > 1) & (jnp.bitwise_and(nk, 1) == par))
                    def _(par=par):
                        wait_scatter(par)
                plsc.subcore_barrier()
                pltpu.sync_copy(accs.at[_ds8(s * 512, 512)],
                                acc_hbm.at[_ds8(b * BLK + s * 512, 512)])
                plsc.subcore_barrier()

            return 0

        lax.fori_loop(0, (NB + 1) // 2, blk_iter, 0)

    return pl.kernel(
        body,
        out_type=jax.ShapeDtypeStruct((NPAD, w), jnp.float32),
        mesh=_mesh(),
        compiler_params=pltpu.CompilerParams(needs_layout_passes=False, use_tc_tiling_on_sc=False),
        scratch_types=[
            pltpu.VMEM((16,), jnp.int32),
            pltpu.VMEM((unit,), jnp.int32),
            pltpu.VMEM((unit,), jnp.int32),
            pltpu.VMEM((unit,), jnp.int32),
            pltpu.VMEM((unit,), jnp.int32),
            pltpu.VMEM((unit,), jnp.int32),
            pltpu.VMEM((unit,), jnp.int32),
            pltpu.VMEM((unit, w), jnp.float32),
            pltpu.VMEM((unit, w), jnp.float32),
            pltpu.VMEM_SHARED((ACC_ROWS, w), jnp.float32),
            pltpu.SemaphoreType.DMA,
            pltpu.SemaphoreType.DMA,
            pltpu.SemaphoreType.DMA,
            pltpu.SemaphoreType.DMA,
            pltpu.SemaphoreType.DMA,
            pltpu.SemaphoreType.DMA,
        ],
    )


def _row_spec(c):
    return pl.BlockSpec((R, c), lambda i: (i, 0))


def _full_spec(shape):
    return pl.BlockSpec(shape, lambda i: tuple(0 for _ in shape))


@functools.cache
def _t0():
    def body(x_ref, da_ref, db_ref, y_ref, dv_ref):
        dv = 1.0 / jnp.sqrt(da_ref[...] + db_ref[...] + 1.0)
        y_ref[...] = x_ref[...] * dv
        dv_ref[...] = dv

    return pl.pallas_call(
        body,
        grid=(N // R,),
        in_specs=[_row_spec(16), _row_spec(1), _row_spec(1)],
        out_specs=[_row_spec(16), _row_spec(1)],
        out_shape=[
            jax.ShapeDtypeStruct((N, 16), jnp.float32),
            jax.ShapeDtypeStruct((N, 1), jnp.float32),
        ],
    )


@functools.cache
def _t_layer(cin, cout):
    def body(a_ref, y_ref, dv_ref, w_ref, b_ref, o_ref):
        dv = dv_ref[...]
        g = dv * (a_ref[...] + y_ref[...])
        h = jnp.maximum(jnp.dot(g, w_ref[...],
                                preferred_element_type=jnp.float32)
                        + b_ref[...], 0.0)
        o_ref[...] = dv * h

    return pl.pallas_call(
        body,
        grid=(N // R,),
        in_specs=[_row_spec(cin), _row_spec(cin), _row_spec(1),
                  _full_spec((cin, cout)), _full_spec((1, cout))],
        out_specs=_row_spec(cout),
        out_shape=jax.ShapeDtypeStruct((N, cout), jnp.float32),
    )


@functools.cache
def _t3():
    def body(a_ref, y_ref, dv_ref, w3_ref, b3_ref, wf1_ref, bf1_ref,
             wf2_ref, bf2_ref, o_ref):
        dv = dv_ref[...]
        g = dv * (a_ref[...] + y_ref[...])
        h3 = jnp.maximum(jnp.dot(g, w3_ref[...],
                                 preferred_element_type=jnp.float32)
                         + b3_ref[...], 0.0)
        t = jnp.maximum(jnp.dot(h3, wf1_ref[...],
                                preferred_element_type=jnp.float32)
                        + bf1_ref[...], 0.0)
        o_ref[...] = jnp.dot(t, wf2_ref[...],
                             preferred_element_type=jnp.float32) + bf2_ref[...]

    return pl.pallas_call(
        body,
        grid=(N // R,),
        in_specs=[_row_spec(128), _row_spec(128), _row_spec(1),
                  _full_spec((128, 128)), _full_spec((1, 128)),
                  _full_spec((128, 64)), _full_spec((1, 64)),
                  _full_spec((64, 2)), _full_spec((1, 2))],
        out_specs=_row_spec(2),
        out_shape=jax.ShapeDtypeStruct((N, 2), jnp.float32),
    )


def kernel(x, edge_index, W1, b1, W2, b2, W3, b3, Wf1, bf1, Wf2, bf2):
    edge_index = edge_index.astype(jnp.int32)
    src = edge_index[0]
    dst = edge_index[1]
    xp = jnp.pad(x, ((0, 0), (0, 13)))
    W1p = jnp.pad(W1, ((0, 13), (0, 0)))

    cnts, degc2 = _k_count()(dst)
    starts, bstart, braw = _k_prefix()(cnts)
    edge_buf = _k_scatter()(src, dst, starts, bstart, braw)

    y0, dinv = _t0()(xp, degc2[0, :N, None], degc2[1, :N, None])
    acc0 = _k_prop(16)(edge_buf, bstart, y0, jnp.zeros((528, 16), jnp.float32))
    y1 = _t_layer(16, 64)(acc0[:N], y0, dinv, W1p, b1[None])
    acc1 = _k_prop(64)(edge_buf, bstart, y1, jnp.zeros((528, 64), jnp.float32))
    y2 = _t_layer(64, 128)(acc1[:N], y1, dinv, W2, b2[None])
    acc2 = _k_prop(128)(edge_buf, bstart, y2,
                        jnp.zeros((528, 128), jnp.float32))
    out = _t3()(acc2[:N], y2, dinv, W3, b3[None], Wf1, bf1[None],
                Wf2, bf2[None])
    return out
